# baseline (speedup 1.0000x reference)
import jax
import jax.numpy as jnp
from jax.experimental import pallas as pl
from jax.experimental.pallas import tpu as pltpu

B, N, D, M, DEG = 256, 64, 256, 256, 16
BB = 8
BBN = BB * N
BF = jnp.bfloat16
F32 = jnp.float32


def _abd_kernel(ids_ref, abd_ref):
    ids = ids_ref[...]
    ids_t = jnp.tile(ids, (BB, 1))
    cmod = jax.lax.broadcasted_iota(jnp.int32, (BBN, BBN), 1) % N
    acc = jnp.zeros((BBN, BBN), F32)
    for j in range(DEG):
        acc += (ids_t[:, j : j + 1] == cmod).astype(F32)
    rblk = jax.lax.broadcasted_iota(jnp.int32, (BBN, BBN), 0) // N
    cblk = jax.lax.broadcasted_iota(jnp.int32, (BBN, BBN), 1) // N
    abd_ref[...] = jnp.where(rblk == cblk, acc * (1.0 / DEG), 0.0).astype(BF)


def _ln(x, g, b, eps=1e-5):
    mu = jnp.mean(x, axis=-1, keepdims=True)
    ms = jnp.mean(x * x, axis=-1, keepdims=True)
    a = jax.lax.rsqrt(ms - mu * mu + eps)
    return (x - mu) * a * g + b


def _dot(a, b):
    return jnp.dot(a, b, preferred_element_type=F32)


def _main_kernel(x_ref, abd_ref, w1_ref, b1_ref, g1_ref, be1_ref,
                 w2_ref, b2_ref, w3a_ref, w3b_ref, b3_ref, g2_ref, be2_ref,
                 w4_ref, out_ref):
    x = x_ref[...].reshape(BBN, D).astype(BF)
    h = _dot(x, w1_ref[...])
    h = jnp.maximum(h + b1_ref[...], 0.0)
    h = _ln(h, g1_ref[...], be1_ref[...]).astype(BF)
    t1 = _dot(h, w2_ref[...]) + b2_ref[...]
    t1b = t1.astype(BF)
    u = _dot(t1b, w3a_ref[...])
    v = _dot(t1b, w3b_ref[...]).astype(BF)
    w = _dot(abd_ref[...], v)
    h2 = jnp.maximum(u + w + b3_ref[...], 0.0)
    h2 = _ln(h2, g2_ref[...], be2_ref[...])
    out_ref[...] = jnp.sum(h2 * w4_ref[...], axis=1, keepdims=True)


def _softmax_kernel(lg_ref, out_ref):
    lg = lg_ref[...]
    e = jnp.exp(lg - jnp.max(lg, axis=1, keepdims=True))
    out_ref[...] = e / jnp.sum(e, axis=1, keepdims=True)


def kernel(inp, ids_list, W1, b1, g1, be1, W2, b2, W3, b3, g2, be2, W4, b4):
    ids32 = ids_list.astype(jnp.int32)
    abd = pl.pallas_call(
        _abd_kernel,
        out_shape=jax.ShapeDtypeStruct((BBN, BBN), BF),
    )(ids32)

    row = lambda v: v.reshape(1, -1)
    full = lambda s: pl.BlockSpec(s, lambda i: (0,) * len(s))
    logits = pl.pallas_call(
        _main_kernel,
        grid=(B // BB,),
        in_specs=[
            pl.BlockSpec((BB, N, D), lambda i: (i, 0, 0)),
            full((BBN, BBN)),
            full((D, M)), full((1, M)), full((1, M)), full((1, M)),
            full((M, M)), full((1, M)),
            full((M, M)), full((M, M)), full((1, M)), full((1, M)), full((1, M)),
            full((1, M)),
        ],
        out_specs=pl.BlockSpec((BBN, 1), lambda i: (i, 0)),
        out_shape=jax.ShapeDtypeStruct((B * N, 1), F32),
        compiler_params=pltpu.CompilerParams(
            dimension_semantics=("parallel",)),
    )(inp, abd, W1.astype(BF), row(b1), row(g1), row(be1),
      W2.astype(BF), row(b2),
      W3[:M].astype(BF), W3[M:].astype(BF), row(b3), row(g2), row(be2),
      W4.reshape(1, M))

    out = pl.pallas_call(
        _softmax_kernel,
        out_shape=jax.ShapeDtypeStruct((B, N), F32),
    )(logits.reshape(B, N))
    return out


# speedup vs baseline: 1.0037x; 1.0037x over previous
import jax
import jax.numpy as jnp
from jax.experimental import pallas as pl
from jax.experimental.pallas import tpu as pltpu

B, N, D, M, DEG = 256, 64, 256, 256, 16
BB = 8
BBN = BB * N
F32 = jnp.float32


def _abd_kernel(ids_ref, abd_ref):
    ids = ids_ref[...]
    ids_t = jnp.tile(ids, (BB, 1))
    cmod = jax.lax.broadcasted_iota(jnp.int32, (BBN, BBN), 1) % N
    acc = jnp.zeros((BBN, BBN), F32)
    for j in range(DEG):
        acc += (ids_t[:, j : j + 1] == cmod).astype(F32)
    rblk = jax.lax.broadcasted_iota(jnp.int32, (BBN, BBN), 0) // N
    cblk = jax.lax.broadcasted_iota(jnp.int32, (BBN, BBN), 1) // N
    abd_ref[...] = jnp.where(rblk == cblk, acc * (1.0 / DEG), 0.0)


def _norm(x, eps=1e-5):
    mu = jnp.mean(x, axis=-1, keepdims=True)
    ms = jnp.mean(x * x, axis=-1, keepdims=True)
    a = jax.lax.rsqrt(ms - mu * mu + eps)
    return (x - mu) * a


def _dot(a, b):
    return jnp.dot(a, b, preferred_element_type=F32)


def _main_kernel(x_ref, abd_ref, w1_ref, b1_ref, w2_ref, b2_ref,
                 w3a_ref, w3b_ref, b3_ref, w4_ref, out_ref):
    x = x_ref[...].reshape(BBN, D)
    h = jnp.maximum(_dot(x, w1_ref[...]) + b1_ref[...], 0.0)
    h = _norm(h)
    t1 = _dot(h, w2_ref[...]) + b2_ref[...]
    u = _dot(t1, w3a_ref[...])
    v = _dot(t1, w3b_ref[...])
    w = _dot(abd_ref[...], v)
    h2 = _norm(jnp.maximum(u + w + b3_ref[...], 0.0))
    out_ref[...] = jnp.sum(h2 * w4_ref[...], axis=1, keepdims=True)


def _softmax_kernel(lg_ref, out_ref):
    lg = lg_ref[...]
    e = jnp.exp(lg - jnp.max(lg, axis=1, keepdims=True))
    out_ref[...] = e / jnp.sum(e, axis=1, keepdims=True)


def kernel(inp, ids_list, W1, b1, g1, be1, W2, b2, W3, b3, g2, be2, W4, b4):
    ids32 = ids_list.astype(jnp.int32)
    abd = pl.pallas_call(
        _abd_kernel,
        out_shape=jax.ShapeDtypeStruct((BBN, BBN), F32),
    )(ids32)

    W2f = g1[:, None] * W2
    b2f = b2 + be1 @ W2
    w4f = (g2 * W4[:, 0]).reshape(1, M)

    row = lambda v: v.reshape(1, -1)
    full = lambda s: pl.BlockSpec(s, lambda i: (0,) * len(s))
    logits = pl.pallas_call(
        _main_kernel,
        grid=(B // BB,),
        in_specs=[
            pl.BlockSpec((BB, N, D), lambda i: (i, 0, 0)),
            full((BBN, BBN)),
            full((D, M)), full((1, M)),
            full((M, M)), full((1, M)),
            full((M, M)), full((M, M)), full((1, M)),
            full((1, M)),
        ],
        out_specs=pl.BlockSpec((BBN, 1), lambda i: (i, 0)),
        out_shape=jax.ShapeDtypeStruct((B * N, 1), F32),
        compiler_params=pltpu.CompilerParams(
            dimension_semantics=("parallel",)),
    )(inp, abd, W1, row(b1), W2f, row(b2f), W3[:M], W3[M:], row(b3), w4f)

    out = pl.pallas_call(
        _softmax_kernel,
        out_shape=jax.ShapeDtypeStruct((B, N), F32),
    )(logits.reshape(B, N))
    return out


# speedup vs baseline: 1.0758x; 1.0718x over previous
import jax
import jax.numpy as jnp
from jax.experimental import pallas as pl
from jax.experimental.pallas import tpu as pltpu

B, N, D, M, DEG = 256, 64, 256, 256, 16
BB = 8
BBN = BB * N
F32 = jnp.float32


def _abd_kernel(ids_ref, abd_ref):
    ids = ids_ref[...]
    ids_t = jnp.tile(ids, (BB, 1))
    cmod = jax.lax.broadcasted_iota(jnp.int32, (BBN, BBN), 1) % N
    acc = jnp.zeros((BBN, BBN), F32)
    for j in range(DEG):
        acc += (ids_t[:, j : j + 1] == cmod).astype(F32)
    rblk = jax.lax.broadcasted_iota(jnp.int32, (BBN, BBN), 0) // N
    cblk = jax.lax.broadcasted_iota(jnp.int32, (BBN, BBN), 1) // N
    abd_ref[...] = jnp.where(rblk == cblk, acc * (1.0 / DEG), 0.0)


def _ln(x, g, b, eps=1e-5):
    mu = jnp.mean(x, axis=-1, keepdims=True)
    ms = jnp.mean(x * x, axis=-1, keepdims=True)
    a = jax.lax.rsqrt(ms - mu * mu + eps)
    return (x - mu) * a * g + b


def _dot(a, b):
    return jnp.dot(a, b, preferred_element_type=F32)


def _main_kernel(x_ref, abd_ref, w1_ref, b1_ref, g1_ref, be1_ref,
                 w2_ref, b2_ref, w3_ref, b3_ref, g2_ref, be2_ref,
                 w4_ref, out_ref):
    x = x_ref[...].reshape(BBN, D)
    h = jnp.maximum(_dot(x, w1_ref[...]) + b1_ref[...], 0.0)
    h = _ln(h, g1_ref[...], be1_ref[...])
    t1 = _dot(h, w2_ref[...]) + b2_ref[...]
    u = _dot(t1, w3_ref[0:M, :])
    v = _dot(t1, w3_ref[M:2 * M, :])
    w = _dot(abd_ref[...], v)
    h2 = jnp.maximum(u + w + b3_ref[...], 0.0)
    h2 = _ln(h2, g2_ref[...], be2_ref[...])
    lrow = jax.lax.dot_general(w4_ref[...], h2, (((1,), (1,)), ((), ())),
                               preferred_element_type=F32)
    lg = jnp.concatenate([lrow[:, b * N:(b + 1) * N] for b in range(BB)],
                         axis=0)
    e = jnp.exp(lg - jnp.max(lg, axis=1, keepdims=True))
    out_ref[...] = e / jnp.sum(e, axis=1, keepdims=True)


def kernel(inp, ids_list, W1, b1, g1, be1, W2, b2, W3, b3, g2, be2, W4, b4):
    ids32 = ids_list.astype(jnp.int32)
    abd = pl.pallas_call(
        _abd_kernel,
        out_shape=jax.ShapeDtypeStruct((BBN, BBN), F32),
    )(ids32)

    row = lambda v: v.reshape(1, -1)
    full = lambda s: pl.BlockSpec(s, lambda i: (0,) * len(s))
    out = pl.pallas_call(
        _main_kernel,
        grid=(B // BB,),
        in_specs=[
            pl.BlockSpec((BB, N, D), lambda i: (i, 0, 0)),
            full((BBN, BBN)),
            full((D, M)), full((1, M)), full((1, M)), full((1, M)),
            full((M, M)), full((1, M)),
            full((2 * M, M)), full((1, M)), full((1, M)), full((1, M)),
            full((1, M)),
        ],
        out_specs=pl.BlockSpec((BB, N), lambda i: (i, 0)),
        out_shape=jax.ShapeDtypeStruct((B, N), F32),
        compiler_params=pltpu.CompilerParams(
            dimension_semantics=("parallel",)),
    )(inp, abd, W1, row(b1), row(g1), row(be1), W2, row(b2),
      W3, row(b3), row(g2), row(be2), W4.reshape(1, M))
    return out


# speedup vs baseline: 1.2198x; 1.1338x over previous
import jax
import jax.numpy as jnp
from jax.experimental import pallas as pl
from jax.experimental.pallas import tpu as pltpu

B, N, D, M, DEG = 256, 64, 256, 256, 16
BB = 16
BBN = BB * N
F32 = jnp.float32


def _abd_kernel(ids_ref, abd_ref):
    ids = ids_ref[...]
    ids_t = jnp.tile(ids, (BB, 1))
    cmod = jax.lax.broadcasted_iota(jnp.int32, (BBN, BBN), 1) % N
    acc = jnp.zeros((BBN, BBN), F32)
    for j in range(DEG):
        acc += (ids_t[:, j : j + 1] == cmod).astype(F32)
    rblk = jax.lax.broadcasted_iota(jnp.int32, (BBN, BBN), 0) // N
    cblk = jax.lax.broadcasted_iota(jnp.int32, (BBN, BBN), 1) // N
    abd_ref[...] = jnp.where(rblk == cblk, acc * (1.0 / DEG), 0.0)


def _ln(x, g, b, eps=1e-5):
    mu = jnp.mean(x, axis=-1, keepdims=True)
    ms = jnp.mean(x * x, axis=-1, keepdims=True)
    a = jax.lax.rsqrt(ms - mu * mu + eps)
    return (x - mu) * a * g + b


def _dot(a, b):
    return jnp.dot(a, b, preferred_element_type=F32)


def _main_kernel(x_ref, abd_ref, w1_ref, b1_ref, g1_ref, be1_ref,
                 w2_ref, b2_ref, w3_ref, b3_ref, g2_ref, be2_ref,
                 w4_ref, out_ref):
    x = x_ref[...].reshape(BBN, D)
    h = jnp.maximum(_dot(x, w1_ref[...]) + b1_ref[...], 0.0)
    h = _ln(h, g1_ref[...], be1_ref[...])
    t1 = _dot(h, w2_ref[...]) + b2_ref[...]
    u = _dot(t1, w3_ref[0:M, :])
    v = _dot(t1, w3_ref[M:2 * M, :])
    w = _dot(abd_ref[...], v)
    h2 = jnp.maximum(u + w + b3_ref[...], 0.0)
    h2 = _ln(h2, g2_ref[...], be2_ref[...])
    lrow = jax.lax.dot_general(w4_ref[...], h2, (((1,), (1,)), ((), ())),
                               preferred_element_type=F32)
    lg = jnp.concatenate([lrow[:, b * N:(b + 1) * N] for b in range(BB)],
                         axis=0)
    e = jnp.exp(lg - jnp.max(lg, axis=1, keepdims=True))
    out_ref[...] = e / jnp.sum(e, axis=1, keepdims=True)


def kernel(inp, ids_list, W1, b1, g1, be1, W2, b2, W3, b3, g2, be2, W4, b4):
    ids32 = ids_list.astype(jnp.int32)
    abd = pl.pallas_call(
        _abd_kernel,
        out_shape=jax.ShapeDtypeStruct((BBN, BBN), F32),
    )(ids32)

    row = lambda v: v.reshape(1, -1)
    full = lambda s: pl.BlockSpec(s, lambda i: (0,) * len(s))
    out = pl.pallas_call(
        _main_kernel,
        grid=(B // BB,),
        in_specs=[
            pl.BlockSpec((BB, N, D), lambda i: (i, 0, 0)),
            full((BBN, BBN)),
            full((D, M)), full((1, M)), full((1, M)), full((1, M)),
            full((M, M)), full((1, M)),
            full((2 * M, M)), full((1, M)), full((1, M)), full((1, M)),
            full((1, M)),
        ],
        out_specs=pl.BlockSpec((BB, N), lambda i: (i, 0)),
        out_shape=jax.ShapeDtypeStruct((B, N), F32),
        compiler_params=pltpu.CompilerParams(
            dimension_semantics=("parallel",)),
    )(inp, abd, W1, row(b1), row(g1), row(be1), W2, row(b2),
      W3, row(b3), row(g2), row(be2), W4.reshape(1, M))
    return out
